# baseline (speedup 1.0000x reference)
import functools

import jax
import jax.numpy as jnp
from jax import lax
from jax.experimental import pallas as pl
from jax.experimental.pallas import tpu as pltpu
from jax.experimental.pallas import tpu_sc as plsc

D = 64
NC, NS = 2, 16
NW = NC * NS
C = 1024
G = 128


def _comb_body(minute_ref, hour_ref, weekday_ref, duration_ref, out_ref):
    hm = (jnp.broadcast_to(hour_ref[:][:, None, :], (24, 4, D))
          + jnp.broadcast_to(minute_ref[:][None, :, :], (24, 4, D))).reshape(96, D)
    row = lax.broadcasted_iota(jnp.int32, (7, D), 0) == pl.program_id(0)
    w = jnp.sum(jnp.where(row, weekday_ref[:], 0.0), axis=0)
    out_ref[0] = (hm[:, None, :] + duration_ref[:][None, :, :]
                  + w[None, None, :])


def _build_comb(minute_W, hour_W, weekday_W, duration_W):
    out = pl.pallas_call(
        _comb_body,
        grid=(7,),
        in_specs=[
            pl.BlockSpec((4, D), lambda w: (0, 0)),
            pl.BlockSpec((24, D), lambda w: (0, 0)),
            pl.BlockSpec((7, D), lambda w: (0, 0)),
            pl.BlockSpec((96, D), lambda w: (0, 0)),
        ],
        out_specs=pl.BlockSpec((1, 96, 96, D), lambda w: (w, 0, 0, 0)),
        out_shape=jax.ShapeDtypeStruct((7, 96, 96, D), jnp.float32),
    )(minute_W, hour_W, weekday_W, duration_W)
    return out.reshape(7 * 96 * 96, D)


def _cidx_body(t_ref, w_ref, d_ref, out_ref):
    out_ref[...] = (w_ref[...] * 96 + t_ref[...]) * 96 + d_ref[...]


def _build_cidx(time, weekday, duration):
    B, L = time.shape
    blk = 512
    return pl.pallas_call(
        _cidx_body,
        grid=(B // blk,),
        in_specs=[pl.BlockSpec((blk, L), lambda i: (i, 0))] * 3,
        out_specs=pl.BlockSpec((blk, L), lambda i: (i, 0)),
        out_shape=jax.ShapeDtypeStruct((B, L), jnp.int32),
    )(time, weekday, duration)


def _tr_table_body(in_ref, out_ref):
    out_ref[...] = in_ref[...].T


def _relayout_table(locT):
    V = locT.shape[1]
    blk = 4096
    return pl.pallas_call(
        _tr_table_body,
        grid=(pl.cdiv(V, blk),),
        in_specs=[pl.BlockSpec((D, blk), lambda i: (0, i))],
        out_specs=pl.BlockSpec((blk, D), lambda i: (i, 0)),
        out_shape=jax.ShapeDtypeStruct((V, D), jnp.float32),
    )(locT)


def _tr_out_body(in_ref, out_ref):
    for l in range(8):
        out_ref[l] = in_ref[:, l, :].T


def _transpose_out(y, B, L):
    y3 = y.reshape(B, L, D)
    return pl.pallas_call(
        _tr_out_body,
        grid=(L // 8,),
        in_specs=[pl.BlockSpec((B, 8, D), lambda l: (0, l, 0))],
        out_specs=pl.BlockSpec((8, D, B), lambda l: (l, 0, 0)),
        out_shape=jax.ShapeDtypeStruct((L, D, B), jnp.float32),
    )(y3)


def _make_sc_lookup(n_tokens):
    rpw = n_tokens // NW
    nchunk = rpw // C
    mesh = plsc.VectorSubcoreMesh(core_axis_name="c", subcore_axis_name="s")

    @functools.partial(
        pl.kernel,
        mesh=mesh,
        out_type=jax.ShapeDtypeStruct((n_tokens, D), jnp.float32),
        scratch_types=[
            pltpu.VMEM((C,), jnp.int32),
            pltpu.VMEM((C,), jnp.int32),
            pltpu.VMEM((C, D), jnp.float32),
            pltpu.SemaphoreType.DMA,
        ],
        compiler_params=pltpu.CompilerParams(use_tc_tiling_on_sc=False,
                                             needs_layout_passes=False),
    )
    def sc_lookup(src_h, cidx_h, comb_h, loc_h, out_h, sbuf, cbuf, rows, sem):
        cid = lax.axis_index("c")
        sid = lax.axis_index("s")
        wid = sid * NC + cid

        @pl.loop(0, nchunk)
        def _chunk(k):
            base = wid * rpw + k * C
            pltpu.sync_copy(src_h.at[pl.ds(base, C)], sbuf)
            pltpu.sync_copy(cidx_h.at[pl.ds(base, C)], cbuf)

            descs = []
            for j in range(C // G):
                descs.append(pltpu.async_copy(
                    loc_h.at[sbuf.at[pl.ds(j * G, G)]],
                    rows.at[pl.ds(j * G, G)], sem))
            for d_ in descs:
                d_.wait()
            descs = []
            for j in range(C // G):
                descs.append(pltpu.async_copy(
                    comb_h.at[cbuf.at[pl.ds(j * G, G)]],
                    rows.at[pl.ds(j * G, G)], sem, add=True))
            for d_ in descs:
                d_.wait()
            pltpu.sync_copy(rows, out_h.at[pl.ds(base, C)])

    return sc_lookup


def kernel(src, time, weekday, duration, emb_loc_W, minute_W, hour_W,
           weekday_W, duration_W):
    B, L = src.shape
    n = B * L
    comb = _build_comb(minute_W, hour_W, weekday_W, duration_W)
    cidx = _build_cidx(time.astype(jnp.int32), weekday.astype(jnp.int32),
                       duration.astype(jnp.int32))
    loc_rm = _relayout_table(jnp.transpose(emb_loc_W))
    y = _make_sc_lookup(n)(
        src.reshape(n).astype(jnp.int32),
        cidx.reshape(n),
        comb,
        loc_rm,
    )
    out = _transpose_out(y, B, L)
    return jnp.transpose(out, (2, 0, 1))


# speedup vs baseline: 1.0411x; 1.0411x over previous
import functools

import jax
import jax.numpy as jnp
from jax import lax
from jax.experimental import pallas as pl
from jax.experimental.pallas import tpu as pltpu
from jax.experimental.pallas import tpu_sc as plsc

D = 64
NC, NS = 2, 16
NW = NC * NS
C = 1024
G = 128


def _comb_body(minute_ref, hour_ref, weekday_ref, duration_ref, out_ref):
    hm = (jnp.broadcast_to(hour_ref[:][:, None, :], (24, 4, D))
          + jnp.broadcast_to(minute_ref[:][None, :, :], (24, 4, D))).reshape(96, D)
    row = lax.broadcasted_iota(jnp.int32, (7, D), 0) == pl.program_id(0)
    w = jnp.sum(jnp.where(row, weekday_ref[:], 0.0), axis=0)
    out_ref[0] = (hm[:, None, :] + duration_ref[:][None, :, :]
                  + w[None, None, :])


def _build_comb(minute_W, hour_W, weekday_W, duration_W):
    out = pl.pallas_call(
        _comb_body,
        grid=(7,),
        in_specs=[
            pl.BlockSpec((4, D), lambda w: (0, 0)),
            pl.BlockSpec((24, D), lambda w: (0, 0)),
            pl.BlockSpec((7, D), lambda w: (0, 0)),
            pl.BlockSpec((96, D), lambda w: (0, 0)),
        ],
        out_specs=pl.BlockSpec((1, 96, 96, D), lambda w: (w, 0, 0, 0)),
        out_shape=jax.ShapeDtypeStruct((7, 96, 96, D), jnp.float32),
    )(minute_W, hour_W, weekday_W, duration_W)
    return out.reshape(7 * 96 * 96, D)


def _cidx_body(t_ref, w_ref, d_ref, out_ref):
    out_ref[...] = (w_ref[...] * 96 + t_ref[...]) * 96 + d_ref[...]


def _build_cidx(time, weekday, duration):
    B, L = time.shape
    blk = 512
    return pl.pallas_call(
        _cidx_body,
        grid=(B // blk,),
        in_specs=[pl.BlockSpec((blk, L), lambda i: (i, 0))] * 3,
        out_specs=pl.BlockSpec((blk, L), lambda i: (i, 0)),
        out_shape=jax.ShapeDtypeStruct((B, L), jnp.int32),
    )(time, weekday, duration)


def _tr_table_body(in_ref, out_ref):
    out_ref[...] = in_ref[...].T


def _relayout_table(locT):
    V = locT.shape[1]
    blk = 4096
    return pl.pallas_call(
        _tr_table_body,
        grid=(pl.cdiv(V, blk),),
        in_specs=[pl.BlockSpec((D, blk), lambda i: (0, i))],
        out_specs=pl.BlockSpec((blk, D), lambda i: (i, 0)),
        out_shape=jax.ShapeDtypeStruct((V, D), jnp.float32),
    )(locT)


def _tr_out_body(in_ref, out_ref):
    for l in range(8):
        out_ref[l] = in_ref[:, l, :].T


def _transpose_out(y, B, L):
    y3 = y.reshape(B, L, D)
    return pl.pallas_call(
        _tr_out_body,
        grid=(L // 8,),
        in_specs=[pl.BlockSpec((B, 8, D), lambda l: (0, l, 0))],
        out_specs=pl.BlockSpec((8, D, B), lambda l: (l, 0, 0)),
        out_shape=jax.ShapeDtypeStruct((L, D, B), jnp.float32),
    )(y3)


def _make_sc_lookup(n_tokens):
    rpw = n_tokens // NW
    nchunk = rpw // C
    mesh = plsc.VectorSubcoreMesh(core_axis_name="c", subcore_axis_name="s")

    @functools.partial(
        pl.kernel,
        mesh=mesh,
        out_type=jax.ShapeDtypeStruct((n_tokens, D), jnp.float32),
        scratch_types=[
            pltpu.VMEM((C,), jnp.int32),
            pltpu.VMEM((C,), jnp.int32),
            pltpu.VMEM((C, D), jnp.float32),
            pltpu.SemaphoreType.DMA,
        ],
        compiler_params=pltpu.CompilerParams(use_tc_tiling_on_sc=False,
                                             needs_layout_passes=False),
    )
    def sc_lookup(src_h, cidx_h, comb_h, loc_h, out_h, sbuf, cbuf, rows, sem):
        cid = lax.axis_index("c")
        sid = lax.axis_index("s")
        wid = sid * NC + cid

        @pl.loop(0, nchunk)
        def _chunk(k):
            base = wid * rpw + k * C
            pltpu.sync_copy(src_h.at[pl.ds(base, C)], sbuf)
            pltpu.sync_copy(cidx_h.at[pl.ds(base, C)], cbuf)

            descs = []
            for j in range(C // G):
                descs.append(pltpu.async_copy(
                    loc_h.at[sbuf.at[pl.ds(j * G, G)]],
                    rows.at[pl.ds(j * G, G)], sem))
            for d_ in descs:
                d_.wait()
            descs = []
            for j in range(C // G):
                descs.append(pltpu.async_copy(
                    comb_h.at[cbuf.at[pl.ds(j * G, G)]],
                    rows.at[pl.ds(j * G, G)], sem, add=True))
            for d_ in descs:
                d_.wait()
            pltpu.sync_copy(rows, out_h.at[pl.ds(base, C)])

    return sc_lookup


def kernel(src, time, weekday, duration, emb_loc_W, minute_W, hour_W,
           weekday_W, duration_W):
    B, L = src.shape
    n = B * L
    comb = _build_comb(minute_W, hour_W, weekday_W, duration_W)
    cidx = _build_cidx(time.astype(jnp.int32), weekday.astype(jnp.int32),
                       duration.astype(jnp.int32))
    loc_rm = _relayout_table(jnp.transpose(emb_loc_W))
    y = _make_sc_lookup(n)(
        src.reshape(n).astype(jnp.int32),
        cidx.reshape(n),
        comb,
        loc_rm,
    )
    return y.reshape(B, L, D)


# speedup vs baseline: 1.1667x; 1.1206x over previous
import functools

import jax
import jax.numpy as jnp
from jax import lax
from jax.experimental import pallas as pl
from jax.experimental.pallas import tpu as pltpu
from jax.experimental.pallas import tpu_sc as plsc

D = 64
NC, NS = 2, 16
NW = NC * NS
C = 1024
G = 128


def _comb_body(minute_ref, hour_ref, weekday_ref, duration_ref, out_ref):
    hm = (jnp.broadcast_to(hour_ref[:][:, None, :], (24, 4, D))
          + jnp.broadcast_to(minute_ref[:][None, :, :], (24, 4, D))).reshape(96, D)
    row = lax.broadcasted_iota(jnp.int32, (7, D), 0) == pl.program_id(0)
    w = jnp.sum(jnp.where(row, weekday_ref[:], 0.0), axis=0)
    out_ref[0] = (hm[:, None, :] + duration_ref[:][None, :, :]
                  + w[None, None, :])


def _build_comb(minute_W, hour_W, weekday_W, duration_W):
    out = pl.pallas_call(
        _comb_body,
        grid=(7,),
        in_specs=[
            pl.BlockSpec((4, D), lambda w: (0, 0)),
            pl.BlockSpec((24, D), lambda w: (0, 0)),
            pl.BlockSpec((7, D), lambda w: (0, 0)),
            pl.BlockSpec((96, D), lambda w: (0, 0)),
        ],
        out_specs=pl.BlockSpec((1, 96, 96, D), lambda w: (w, 0, 0, 0)),
        out_shape=jax.ShapeDtypeStruct((7, 96, 96, D), jnp.float32),
    )(minute_W, hour_W, weekday_W, duration_W)
    return out.reshape(7 * 96 * 96, D)


def _cidx_body(t_ref, w_ref, d_ref, out_ref):
    out_ref[...] = (w_ref[...] * 96 + t_ref[...]) * 96 + d_ref[...]


def _build_cidx(time, weekday, duration):
    B, L = time.shape
    blk = 512
    return pl.pallas_call(
        _cidx_body,
        grid=(B // blk,),
        in_specs=[pl.BlockSpec((blk, L), lambda i: (i, 0))] * 3,
        out_specs=pl.BlockSpec((blk, L), lambda i: (i, 0)),
        out_shape=jax.ShapeDtypeStruct((B, L), jnp.int32),
    )(time, weekday, duration)


def _tr_table_body(in_ref, out_ref):
    out_ref[...] = in_ref[...].T


def _relayout_table(locT):
    V = locT.shape[1]
    blk = 4096
    return pl.pallas_call(
        _tr_table_body,
        grid=(pl.cdiv(V, blk),),
        in_specs=[pl.BlockSpec((D, blk), lambda i: (0, i))],
        out_specs=pl.BlockSpec((blk, D), lambda i: (i, 0)),
        out_shape=jax.ShapeDtypeStruct((V, D), jnp.float32),
    )(locT)


def _tr_out_body(in_ref, out_ref):
    for l in range(8):
        out_ref[l] = in_ref[:, l, :].T


def _transpose_out(y, B, L):
    y3 = y.reshape(B, L, D)
    return pl.pallas_call(
        _tr_out_body,
        grid=(L // 8,),
        in_specs=[pl.BlockSpec((B, 8, D), lambda l: (0, l, 0))],
        out_specs=pl.BlockSpec((8, D, B), lambda l: (l, 0, 0)),
        out_shape=jax.ShapeDtypeStruct((L, D, B), jnp.float32),
    )(y3)


def _make_sc_lookup(n_tokens):
    rpw = n_tokens // NW
    nchunk = rpw // C
    mesh = plsc.VectorSubcoreMesh(core_axis_name="c", subcore_axis_name="s")

    @functools.partial(
        pl.kernel,
        mesh=mesh,
        out_type=jax.ShapeDtypeStruct((n_tokens, D), jnp.float32),
        scratch_types=[
            pltpu.VMEM((C,), jnp.int32),
            pltpu.VMEM((C,), jnp.int32),
            pltpu.VMEM((C, D), jnp.float32),
            pltpu.SemaphoreType.DMA,
        ],
        compiler_params=pltpu.CompilerParams(use_tc_tiling_on_sc=False,
                                             needs_layout_passes=False),
    )
    def sc_lookup(src_h, cidx_h, comb_h, loc_h, out_h, sbuf, cbuf, rows, sem):
        cid = lax.axis_index("c")
        sid = lax.axis_index("s")
        wid = sid * NC + cid

        @pl.loop(0, nchunk)
        def _chunk(k):
            base = wid * rpw + k * C
            pltpu.sync_copy(src_h.at[pl.ds(base, C)], sbuf)
            pltpu.sync_copy(cidx_h.at[pl.ds(base, C)], cbuf)

            descs = []
            for j in range(C // G):
                descs.append(pltpu.async_copy(
                    loc_h.at[sbuf.at[pl.ds(j * G, G)]],
                    rows.at[pl.ds(j * G, G)], sem))
            for d_ in descs:
                d_.wait()
            descs = []
            for j in range(C // G):
                descs.append(pltpu.async_copy(
                    comb_h.at[cbuf.at[pl.ds(j * G, G)]],
                    rows.at[pl.ds(j * G, G)], sem, add=True))
            for d_ in descs:
                d_.wait()
            pltpu.sync_copy(rows, out_h.at[pl.ds(base, C)])

    return sc_lookup


def kernel(src, time, weekday, duration, emb_loc_W, minute_W, hour_W,
           weekday_W, duration_W):
    B, L = src.shape
    n = B * L
    comb = _build_comb(minute_W, hour_W, weekday_W, duration_W)
    cidx = _build_cidx(time.astype(jnp.int32), weekday.astype(jnp.int32),
                       duration.astype(jnp.int32))
    y = _make_sc_lookup(n)(
        src.reshape(n).astype(jnp.int32),
        cidx.reshape(n),
        comb,
        emb_loc_W,
    )
    return y.reshape(B, L, D)
